# Initial kernel scaffold; baseline (speedup 1.0000x reference)
import jax
import jax.numpy as jnp
from jax.experimental import pallas as pl


def kernel(x, edge_index, batch, W0, a_src0, a_dst0, b0, W1, a_src1, a_dst1, b1, Wp1, bp1, Wp2, bp2):
    raise NotImplementedError("write your pallas kernel here")


# speedup vs baseline: 22.1934x; 22.1934x over previous
import jax
import jax.numpy as jnp
from jax import lax
from jax.experimental import pallas as pl
from jax.experimental.pallas import tpu as pltpu
from jax.experimental.pallas import tpu_sc as plsc

N = 10000
NP = 10240
E = 320000
D = 128
G = 16
NC = 2
NS = 16
NW = NC * NS
EPW = E // NW
CH = 80
NCH = EPW // CH
RPT = NP // NS
BR = 1024
NB = NP // BR


def _mm_alpha_body(x_ref, w_ref, asr_ref, adr_ref, h_ref, oas_ref, oad_ref):
    h = jnp.dot(x_ref[...], w_ref[...], preferred_element_type=jnp.float32)
    h_ref[...] = h
    oas_ref[...] = jnp.sum(h * asr_ref[...], axis=1).reshape(1, 1, BR)
    oad_ref[...] = jnp.sum(h * adr_ref[...], axis=1).reshape(1, 1, BR)


def _mm_alpha(x, w, a_s, a_d):
    return pl.pallas_call(
        _mm_alpha_body,
        grid=(NB,),
        in_specs=[pl.BlockSpec((BR, D), lambda i: (i, 0)),
                  pl.BlockSpec((D, D), lambda i: (0, 0)),
                  pl.BlockSpec((1, D), lambda i: (0, 0)),
                  pl.BlockSpec((1, D), lambda i: (0, 0))],
        out_specs=[pl.BlockSpec((BR, D), lambda i: (i, 0)),
                   pl.BlockSpec((1, 1, BR), lambda i: (i, 0, 0)),
                   pl.BlockSpec((1, 1, BR), lambda i: (i, 0, 0))],
        out_shape=[jax.ShapeDtypeStruct((NP, D), jnp.float32),
                   jax.ShapeDtypeStruct((NB, 1, BR), jnp.float32),
                   jax.ShapeDtypeStruct((NB, 1, BR), jnp.float32)],
    )(x, w, a_s, a_d)


def _edge_body(h_hbm, as_hbm, ad_hbm, src_hbm, dst_hbm,
               psum_hbm, pden_hbm,
               as_v, ad_v, den_v, ex_v, rows_v,
               sidx_v, didx_v, psum_sh, sem):
    cid = lax.axis_index("c")
    sid = lax.axis_index("s")
    wid = cid * NS + sid
    z16 = jnp.zeros((16,), jnp.float32)

    pltpu.sync_copy(as_hbm, as_v)
    pltpu.sync_copy(ad_hbm, ad_v)

    def zden(i, c):
        den_v[pl.ds(i * 16, 16)] = z16
        return c
    lax.fori_loop(0, NP // 16, zden, 0)

    def zrows(i, c):
        rows_v[i // (D // 16), pl.ds((i % (D // 16)) * 16, 16)] = z16
        return c
    lax.fori_loop(0, CH * (D // 16), zrows, 0)
    base = sid * RPT
    for k in range(RPT // CH):
        pltpu.sync_copy(rows_v, psum_sh.at[pl.ds(base + k * CH, CH), :])
    plsc.subcore_barrier()

    def p2(c, carry):
        eoff = wid * EPW + c * CH
        pltpu.sync_copy(src_hbm.at[pl.ds(eoff, CH)], sidx_v)
        pltpu.sync_copy(dst_hbm.at[pl.ds(eoff, CH)], didx_v)
        cp = pltpu.async_copy(h_hbm.at[sidx_v], rows_v, sem)
        for j in range(CH // 16):
            s16 = sidx_v[pl.ds(j * 16, 16)]
            d16 = didx_v[pl.ds(j * 16, 16)]
            a = plsc.load_gather(as_v, [s16]) + plsc.load_gather(ad_v, [d16])
            a = jnp.where(a >= 0.0, a, a * jnp.float32(0.2))
            ex = jnp.exp(a)
            ex_v[pl.ds(j * 16, 16)] = ex
            plsc.addupdate_scatter(den_v, [d16], ex)
        cp.wait()

        def scale(r, cc):
            exs = plsc.load_gather(ex_v, [jnp.full((16,), r, jnp.int32)])
            for kk in range(D // 16):
                rows_v[r, pl.ds(kk * 16, 16)] = rows_v[r, pl.ds(kk * 16, 16)] * exs
            return cc
        lax.fori_loop(0, CH, scale, 0)
        pltpu.sync_copy(rows_v, psum_sh.at[didx_v], add=True)
        return carry
    lax.fori_loop(0, NCH, p2, 0)

    pltpu.sync_copy(den_v, pden_hbm.at[wid])
    plsc.subcore_barrier()
    pltpu.sync_copy(psum_sh.at[pl.ds(base, RPT), :],
                    psum_hbm.at[cid, pl.ds(base, RPT), :])


def _edge(h, asv, adv, src, dst):
    mesh = plsc.VectorSubcoreMesh(core_axis_name="c", subcore_axis_name="s")
    return pl.kernel(
        _edge_body,
        out_type=[jax.ShapeDtypeStruct((NC, NP, D), jnp.float32),
                  jax.ShapeDtypeStruct((NW, NP), jnp.float32)],
        mesh=mesh,
        compiler_params=pltpu.CompilerParams(needs_layout_passes=False),
        scratch_types=[pltpu.VMEM((NP,), jnp.float32),
                       pltpu.VMEM((NP,), jnp.float32),
                       pltpu.VMEM((NP,), jnp.float32),
                       pltpu.VMEM((CH,), jnp.float32),
                       pltpu.VMEM((CH, D), jnp.float32),
                       pltpu.VMEM((CH,), jnp.int32),
                       pltpu.VMEM((CH,), jnp.int32),
                       pltpu.VMEM_SHARED((NP, D), jnp.float32),
                       pltpu.SemaphoreType.DMA],
    )(h, asv, adv, src, dst)


def _comb_mm_body(ps_ref, pd_ref, b_ref, w_ref, asr_ref, adr_ref,
                  h_ref, oas_ref, oad_ref):
    p = ps_ref[0] + ps_ref[1]
    den = jnp.sum(pd_ref[...], axis=0)
    x1 = jnp.maximum(p / (den[:, None] + 1e-16) + b_ref[...], 0.0)
    h = jnp.dot(x1, w_ref[...], preferred_element_type=jnp.float32)
    h_ref[...] = h
    oas_ref[...] = jnp.sum(h * asr_ref[...], axis=1).reshape(1, 1, BR)
    oad_ref[...] = jnp.sum(h * adr_ref[...], axis=1).reshape(1, 1, BR)


def _comb_mm(ps, pd, b, w, a_s, a_d):
    return pl.pallas_call(
        _comb_mm_body,
        grid=(NB,),
        in_specs=[pl.BlockSpec((NC, BR, D), lambda i: (0, i, 0)),
                  pl.BlockSpec((NW, BR), lambda i: (0, i)),
                  pl.BlockSpec((1, D), lambda i: (0, 0)),
                  pl.BlockSpec((D, D), lambda i: (0, 0)),
                  pl.BlockSpec((1, D), lambda i: (0, 0)),
                  pl.BlockSpec((1, D), lambda i: (0, 0))],
        out_specs=[pl.BlockSpec((BR, D), lambda i: (i, 0)),
                   pl.BlockSpec((1, 1, BR), lambda i: (i, 0, 0)),
                   pl.BlockSpec((1, 1, BR), lambda i: (i, 0, 0))],
        out_shape=[jax.ShapeDtypeStruct((NP, D), jnp.float32),
                   jax.ShapeDtypeStruct((NB, 1, BR), jnp.float32),
                   jax.ShapeDtypeStruct((NB, 1, BR), jnp.float32)],
    )(ps, pd, b, w, a_s, a_d)


def _pool_body(ps_ref, pd_ref, b_ref, batch_ref, wp1_ref, bp1_ref,
               wp2_ref, bp2_ref, out_ref, acc, cnt):
    i = pl.program_id(0)

    @pl.when(i == 0)
    def _():
        acc[...] = jnp.zeros_like(acc)
        cnt[...] = jnp.zeros_like(cnt)

    p = ps_ref[0] + ps_ref[1]
    den = jnp.sum(pd_ref[...], axis=0)
    h2 = jnp.maximum(p / (den[:, None] + 1e-16) + b_ref[...], 0.0)
    bb = batch_ref[...].reshape(1, BR)
    iot = lax.broadcasted_iota(jnp.int32, (G, BR), 0)
    oh = (iot == bb).astype(jnp.float32)
    acc[...] += lax.dot_general(oh, h2, (((1,), (0,)), ((), ())),
                                preferred_element_type=jnp.float32)
    cnt[...] += jnp.dot(oh, jnp.ones((BR, D), jnp.float32),
                        preferred_element_type=jnp.float32)

    @pl.when(i == pl.num_programs(0) - 1)
    def _():
        pooled = acc[...] / jnp.maximum(cnt[...], 1.0)
        z = jnp.maximum(jnp.dot(pooled, wp1_ref[...],
                                preferred_element_type=jnp.float32)
                        + bp1_ref[...], 0.0)
        out_ref[...] = (jnp.dot(z, wp2_ref[...],
                                preferred_element_type=jnp.float32)
                        + bp2_ref[...])


def _pool(ps, pd, b, batch3, wp1, bp1, wp2p, bp2p):
    return pl.pallas_call(
        _pool_body,
        grid=(NB,),
        in_specs=[pl.BlockSpec((NC, BR, D), lambda i: (0, i, 0)),
                  pl.BlockSpec((NW, BR), lambda i: (0, i)),
                  pl.BlockSpec((1, D), lambda i: (0, 0)),
                  pl.BlockSpec((1, 1, BR), lambda i: (i, 0, 0)),
                  pl.BlockSpec((D, D), lambda i: (0, 0)),
                  pl.BlockSpec((1, D), lambda i: (0, 0)),
                  pl.BlockSpec((D, D), lambda i: (0, 0)),
                  pl.BlockSpec((1, D), lambda i: (0, 0))],
        out_specs=pl.BlockSpec((G, D), lambda i: (0, 0)),
        out_shape=jax.ShapeDtypeStruct((G, D), jnp.float32),
        scratch_shapes=[pltpu.VMEM((G, D), jnp.float32),
                        pltpu.VMEM((G, D), jnp.float32)],
    )(ps, pd, b, batch3, wp1, bp1, wp2p, bp2p)


def kernel(x, edge_index, batch, W0, a_src0, a_dst0, b0,
           W1, a_src1, a_dst1, b1, Wp1, bp1, Wp2, bp2):
    xp = jnp.pad(x, ((0, NP - N), (0, 0)))
    batch3 = jnp.pad(batch, (0, NP - N), constant_values=G).reshape(NB, 1, BR)
    src = edge_index[0]
    dst = edge_index[1]

    h0, as0, ad0 = _mm_alpha(xp, W0, a_src0, a_dst0)
    ps0, pd0 = _edge(h0, as0.reshape(NP), ad0.reshape(NP), src, dst)
    h1, as1, ad1 = _comb_mm(ps0, pd0, b0.reshape(1, D), W1, a_src1, a_dst1)
    ps1, pd1 = _edge(h1, as1.reshape(NP), ad1.reshape(NP), src, dst)

    wp2p = jnp.pad(Wp2, ((0, 0), (0, D - 1)))
    bp2p = jnp.pad(bp2, (0, D - 1)).reshape(1, D)
    out = _pool(ps1, pd1, b1.reshape(1, D), batch3,
                Wp1, bp1.reshape(1, D), wp2p, bp2p)
    return out[:, :1]
